# baseline (speedup 1.0000x reference)
import jax
import jax.numpy as jnp
from jax.experimental import pallas as pl
from jax.experimental.pallas import tpu as pltpu


def _lin(x, W, b):
    return x @ W.T + b


def _gcn(h, src, dst, ew, W, b, n):
    hw = h @ W.T
    loop = jnp.arange(n)
    s = jnp.concatenate([src, loop])
    d = jnp.concatenate([dst, loop])
    w = jnp.concatenate([ew, jnp.ones((n,), ew.dtype)])
    deg = jnp.zeros((n,), hw.dtype).at[d].add(w)
    dinv = jax.lax.rsqrt(jnp.where(deg > 0, deg, 1.0))
    dinv = jnp.where(deg > 0, dinv, 0.0)
    norm = dinv[s] * w * dinv[d]
    out = jnp.zeros_like(hw).at[d].add(hw[s] * norm[:, None])
    return out + b


def _head_kernel(he_ref, hf_ref, hg_ref,
                 co_W1_ref, co_b1_ref, co_A0_ref, co_A1_ref, co_b2_ref,
                 g_W1_ref, g_b1_ref, g_A_ref, g_b2_ref,
                 cls_W1p_ref, cls_b1p_ref, cls_W2p_ref, cls_b2p_ref,
                 pe_W1_ref, pe_b1_ref, pe_W2_ref, pe_b2_ref,
                 pf_W1_ref, pf_b1_ref, pf_W2_ref, pf_b2_ref,
                 logits_ref, ze_ref, zf_ref):
    h_e = he_ref[...]
    h_f = hf_ref[...]
    h_global = hg_ref[...]
    ones_hh = jnp.ones((128, 128), jnp.float32)
    co_in = jnp.concatenate([h_e, h_f, h_e * h_f], axis=-1)
    a1 = jnp.maximum(co_in @ co_W1_ref[...].T + co_b1_ref[...], 0.0)
    alpha0 = jax.nn.sigmoid(a1 @ co_A0_ref[...] + co_b2_ref[0:1, :])
    alpha1 = jax.nn.sigmoid(a1 @ co_A1_ref[...] + co_b2_ref[1:2, :])
    s = alpha0 + alpha1 + 1e-6
    h_cross = (alpha0 * h_e + alpha1 * h_f) / s
    g1 = jnp.maximum(jnp.concatenate([h_cross, h_global], axis=-1) @ g_W1_ref[...].T + g_b1_ref[...], 0.0)
    gg = jax.nn.sigmoid(g1 @ g_A_ref[...] + g_b2_ref[...])
    h_final = gg * h_cross + (1.0 - gg) * h_global
    c1 = jnp.maximum(h_final @ cls_W1p_ref[...].T + cls_b1p_ref[...], 0.0)
    logits_ref[...] = c1 @ cls_W2p_ref[...].T + cls_b2p_ref[...]
    e1 = jnp.maximum(h_e @ pe_W1_ref[...].T + pe_b1_ref[...], 0.0)
    ze = e1 @ pe_W2_ref[...].T + pe_b2_ref[...]
    sse = (ze * ze) @ ones_hh
    ze_ref[...] = ze / jnp.maximum(jnp.sqrt(sse), 1e-12)
    f1 = jnp.maximum(h_f @ pf_W1_ref[...].T + pf_b1_ref[...], 0.0)
    zf = f1 @ pf_W2_ref[...].T + pf_b2_ref[...]
    ssf = (zf * zf) @ ones_hh
    zf_ref[...] = zf / jnp.maximum(jnp.sqrt(ssf), 1e-12)


def kernel(x, edge_index, edge_weight, batch, eeg_mask, eeg_W, eeg_b, fnirs_W, fnirs_b, em_W1, em_b1, em_W2, em_b2, intra_W0, intra_b0, cross_W0, cross_b0, gate0, bn_g0, bn_b0, intra_W1, intra_b1, cross_W1, cross_b1, gate1, bn_g1, bn_b1, co_W1, co_b1, co_W2, co_b2, g_W1, g_b1, g_W2, g_b2, cls_W1, cls_b1, cls_W2, cls_b2, pe_W1, pe_b1, pe_W2, pe_b2, pf_W1, pf_b1, pf_W2, pf_b2):
    B = 64
    mask = eeg_mask.astype(bool)
    mf = mask.astype(jnp.float32)
    n = x.shape[0]
    he = jax.nn.relu(_lin(x, eeg_W, eeg_b))
    hfp = jax.nn.relu(_lin(x, fnirs_W, fnirs_b))
    h = jnp.where(mask[:, None], he, hfp)
    src = edge_index[0]
    dst = edge_index[1]
    diff = jnp.abs(h[src] - h[dst])
    feat = jnp.concatenate([diff, edge_weight[:, None]], axis=1)
    gmod = jax.nn.sigmoid(_lin(jax.nn.relu(_lin(feat, em_W1, em_b1)), em_W2, em_b2))[:, 0]
    ew_hat = edge_weight * jnp.clip(gmod, 0.2, 1.2)
    intra = (mask[src] == mask[dst]).astype(jnp.float32)
    ew_i = ew_hat * intra
    ew_c = ew_hat * (1.0 - intra)
    for (Wi, bi, Wc, bc, gt, bg, bb) in [
        (intra_W0, intra_b0, cross_W0, cross_b0, gate0, bn_g0, bn_b0),
        (intra_W1, intra_b1, cross_W1, cross_b1, gate1, bn_g1, bn_b1)]:
        hi = _gcn(h, src, dst, ew_i, Wi, bi, n)
        hc = _gcn(h, src, dst, ew_c, Wc, bc, n)
        h = hi + jax.nn.sigmoid(gt) * hc
        h = h / jnp.sqrt(1.0 + 1e-5) * bg + bb
        h = jax.nn.relu(h)
    ones = jnp.ones((n,), jnp.float32)
    cnt_all = jnp.maximum(jax.ops.segment_sum(ones, batch, B), 1.0)[:, None]
    h_global = jax.ops.segment_sum(h, batch, B) / cnt_all
    cnt_e = jnp.maximum(jax.ops.segment_sum(mf, batch, B), 1.0)[:, None]
    h_e = jax.ops.segment_sum(h * mf[:, None], batch, B) / cnt_e
    cnt_f = jnp.maximum(jax.ops.segment_sum(1.0 - mf, batch, B), 1.0)[:, None]
    h_f = jax.ops.segment_sum(h * (1.0 - mf)[:, None], batch, B) / cnt_f

    co_A0 = jnp.broadcast_to(co_W2[0][:, None], (128, 128))
    co_A1 = jnp.broadcast_to(co_W2[1][:, None], (128, 128))
    co_b2r = jnp.broadcast_to(co_b2[:, None], (2, 128))
    g_A = jnp.broadcast_to(g_W2[0][:, None], (128, 128))
    g_b2r = jnp.broadcast_to(g_b2[:, None], (1, 128))
    cls_W1p = jnp.zeros((128, 128), jnp.float32).at[:64, :].set(cls_W1)
    cls_b1p = jnp.zeros((128,), jnp.float32).at[:64].set(cls_b1)
    cls_W2p = jnp.zeros((128, 128), jnp.float32).at[:2, :64].set(cls_W2)
    cls_b2p = jnp.zeros((128,), jnp.float32).at[:2].set(cls_b2)

    out_shapes = (
        jax.ShapeDtypeStruct((B, 128), jnp.float32),
        jax.ShapeDtypeStruct((B, 128), jnp.float32),
        jax.ShapeDtypeStruct((B, 128), jnp.float32),
    )
    logits_p, ze, zf = pl.pallas_call(
        _head_kernel,
        out_shape=out_shapes,
    )(h_e, h_f, h_global,
      co_W1, co_b1, co_A0, co_A1, co_b2r,
      g_W1, g_b1, g_A, g_b2r,
      cls_W1p, cls_b1p, cls_W2p, cls_b2p,
      pe_W1, pe_b1, pe_W2, pe_b2,
      pf_W1, pf_b1, pf_W2, pf_b2)
    return (logits_p[:, :2], ze, zf)


# speedup vs baseline: 9.8971x; 9.8971x over previous
import functools

import jax
import jax.numpy as jnp
from jax import lax
from jax.experimental import pallas as pl
from jax.experimental.pallas import tpu as pltpu, tpu_sc as plsc

N = 10000
E = 320000
D = 128
H = 128
B = 64

NPAD = 10240
NN2 = 2 * NPAD
NW = 32
CH = 128
NCHA = 79
EA = NW * NCHA * CH
NCHD = 84
ED = NW * NCHD * CH
LA = NCHA * CH
LD = NCHD * CH

_f32 = jnp.float32
_i32 = jnp.int32

def _wid():
    return lax.axis_index("s") * 2 + lax.axis_index("c")


def _reg_splat(vec, i):
    dn = lax.GatherDimensionNumbers(offset_dims=(), collapsed_slice_dims=(0,),
                                    start_index_map=(0,))
    return lax.gather(vec, jnp.full((16, 1), i, _i32), dn, (1,),
                      mode=lax.GatherScatterMode.PROMISE_IN_BOUNDS)


def _proj_body(x_ref, mf_ref, we_ref, be_ref, wf_ref, bf_ref, o_ref):
    xb = x_ref[...]
    he = jnp.maximum(xb @ we_ref[...].T + be_ref[...], 0.0)
    hf = jnp.maximum(xb @ wf_ref[...].T + bf_ref[...], 0.0)
    h = hf + mf_ref[...] * (he - hf)
    o_ref[0] = h
    o_ref[1] = -h


def _edge_mlp_body(sd_ref, ew_ref, w1_ref, wl_ref, b1_ref, a2_ref, b2_ref, o_ref):
    sd = jnp.abs(sd_ref[...])
    ew = ew_ref[...]
    hid = jnp.maximum(sd @ w1_ref[...].T + ew * wl_ref[...] + b1_ref[...], 0.0)
    t = jax.nn.sigmoid(hid @ a2_ref[...] + b2_ref[...])
    ghat = ew * jnp.clip(t, 0.2, 1.2)
    o_ref[...] = jnp.min(ghat, axis=-1, keepdims=True)


def _dinv_body(degp_ref, o_ref):
    deg = degp_ref[0] + degp_ref[1]
    o_ref[...] = jnp.where(deg > 0.0, lax.rsqrt(jnp.where(deg > 0.0, deg, 1.0)), 0.0)


def _table_body(h_ref, wi_ref, wcs_ref, o_ref):
    hb = h_ref[...]
    o_ref[0] = hb @ wi_ref[...].T
    o_ref[1] = hb @ wcs_ref[...].T


def _combine_body(acc_ref, scale_ref, bias_ref, o_ref):
    s = acc_ref[0] + acc_ref[1]
    o_ref[...] = jnp.maximum(s * scale_ref[...] + bias_ref[...], 0.0)


def _pool_body(h_ref, bt_ref, mf_ref, sa_ref, se_ref, ca_ref, ce_ref):
    i = pl.program_id(0)

    @pl.when(i == 0)
    def _():
        sa_ref[...] = jnp.zeros_like(sa_ref)
        se_ref[...] = jnp.zeros_like(se_ref)
        ca_ref[...] = jnp.zeros_like(ca_ref)
        ce_ref[...] = jnp.zeros_like(ce_ref)

    hb = h_ref[...]
    gids = lax.broadcasted_iota(_i32, (B, 512), 0)
    P = (bt_ref[0] == gids).astype(_f32)
    Pe = P * mf_ref[0]
    ones = jnp.ones((512, 128), _f32)
    sa_ref[...] += P @ hb
    se_ref[...] += Pe @ hb
    ca_ref[...] += P @ ones
    ce_ref[...] += Pe @ ones


def _head_body(sa_ref, se_ref, ca_ref, ce_ref,
               co_W1_ref, co_b1_ref, co_A0_ref, co_A1_ref, co_b2_ref,
               g_W1_ref, g_b1_ref, g_A_ref, g_b2_ref,
               cls_W1p_ref, cls_b1p_ref, cls_W2p_ref, cls_b2p_ref,
               pe_W1_ref, pe_b1_ref, pe_W2_ref, pe_b2_ref,
               pf_W1_ref, pf_b1_ref, pf_W2_ref, pf_b2_ref,
               logits_ref, ze_ref, zf_ref):
    sa = sa_ref[...]
    se = se_ref[...]
    ca = ca_ref[...]
    ce = ce_ref[...]
    h_global = sa / jnp.maximum(ca, 1.0)
    h_e = se / jnp.maximum(ce, 1.0)
    h_f = (sa - se) / jnp.maximum(ca - ce, 1.0)
    ones_hh = jnp.ones((128, 128), _f32)
    co_in = jnp.concatenate([h_e, h_f, h_e * h_f], axis=-1)
    a1 = jnp.maximum(co_in @ co_W1_ref[...].T + co_b1_ref[...], 0.0)
    alpha0 = jax.nn.sigmoid(a1 @ co_A0_ref[...] + co_b2_ref[0:1, :])
    alpha1 = jax.nn.sigmoid(a1 @ co_A1_ref[...] + co_b2_ref[1:2, :])
    s = alpha0 + alpha1 + 1e-6
    h_cross = (alpha0 * h_e + alpha1 * h_f) / s
    g1 = jnp.maximum(
        jnp.concatenate([h_cross, h_global], axis=-1) @ g_W1_ref[...].T + g_b1_ref[...], 0.0)
    gg = jax.nn.sigmoid(g1 @ g_A_ref[...] + g_b2_ref[...])
    h_final = gg * h_cross + (1.0 - gg) * h_global
    c1 = jnp.maximum(h_final @ cls_W1p_ref[...].T + cls_b1p_ref[...], 0.0)
    logits_ref[...] = c1 @ cls_W2p_ref[...].T + cls_b2p_ref[...]
    e1 = jnp.maximum(h_e @ pe_W1_ref[...].T + pe_b1_ref[...], 0.0)
    ze = e1 @ pe_W2_ref[...].T + pe_b2_ref[...]
    ze_ref[...] = ze / jnp.maximum(jnp.sqrt((ze * ze) @ ones_hh), 1e-12)
    f1 = jnp.maximum(h_f @ pf_W1_ref[...].T + pf_b1_ref[...], 0.0)
    zf = f1 @ pf_W2_ref[...].T + pf_b2_ref[...]
    zf_ref[...] = zf / jnp.maximum(jnp.sqrt((zf * zf) @ ones_hh), 1e-12)


def _full(shape):
    return pl.BlockSpec(shape, lambda *_: tuple(0 for _ in shape))


def _sc_edge_prep_body(td, srcA, dstnA, mi2, sdiff, gidxO, ridxO,
                       sv, dv, msb, mdb, gb, rb, rowbuf, sem):
    w = _wid()
    pltpu.sync_copy(srcA.at[w], sv)
    pltpu.sync_copy(dstnA.at[w], dv)
    pltpu.async_copy(mi2.at[sv], msb, sem).wait()
    pltpu.async_copy(mi2.at[dv], mdb, sem).wait()

    def vgrp(g, _):
        sl = pl.ds(g * 16, 16)
        s16 = sv[sl]
        d16 = dv[sl] - NPAD
        cross = msb[sl] != mdb[sl]
        off = jnp.where(cross, NPAD, 0).astype(_i32)
        gb[sl] = s16 + off
        rb[sl] = d16 + off
        return _
    lax.fori_loop(0, LA // 16, vgrp, 0)
    pltpu.sync_copy(gb, gidxO.at[w])
    pltpu.sync_copy(rb, ridxO.at[w])

    def chunk(j, _):
        pltpu.async_copy(td.at[sv.at[pl.ds(j * CH, CH)]], rowbuf, sem).wait()
        pltpu.async_copy(td.at[dv.at[pl.ds(j * CH, CH)]], rowbuf, sem, add=True).wait()
        pltpu.sync_copy(rowbuf, sdiff.at[pl.ds((w * NCHA + j) * CH, CH)])
        return _
    lax.fori_loop(0, NCHA, chunk, 0)


def _sc_deg_body(ridxD, ghatD, degp, rv, wv, zv, acc):
    cid = lax.axis_index("c")
    sid = lax.axis_index("s")
    w = sid * 2 + cid
    pltpu.sync_copy(ridxD.at[w], rv)
    pltpu.sync_copy(ghatD.at[w], wv)
    zero = jnp.zeros((16,), _f32)
    for kk in range(CH // 16):
        zv[pl.ds(kk * 16, 16)] = zero
    per = NN2 // 16
    for b in range(per // CH):
        pltpu.sync_copy(zv, acc.at[pl.ds(sid * per + b * CH, CH)])
    plsc.subcore_barrier()
    pltpu.sync_copy(wv, acc.at[rv], add=True)
    plsc.subcore_barrier()
    pltpu.sync_copy(acc.at[pl.ds(sid * per, per)], degp.at[cid].at[pl.ds(sid * per, per)])


def _sc_coef_body(gidxD, ridxD, ghatD, dinv2, coefD, gv, rv, wv, dg, dr, sem):
    w = _wid()
    pltpu.sync_copy(gidxD.at[w], gv)
    pltpu.sync_copy(ridxD.at[w], rv)
    pltpu.sync_copy(ghatD.at[w], wv)
    pltpu.async_copy(dinv2.at[gv], dg, sem).wait()
    pltpu.async_copy(dinv2.at[rv], dr, sem).wait()

    def vgrp(g, _):
        sl = pl.ds(g * 16, 16)
        wv[sl] = wv[sl] * dg[sl] * dr[sl]
        return _
    lax.fori_loop(0, LD // 16, vgrp, 0)
    pltpu.sync_copy(wv, coefD.at[w])


def _sc_scatter_body(tbl, gidxD, dstDf, coefD, outp, gvc, dvc, cvc, rowbuf, acc, sem):
    cid = lax.axis_index("c")
    sid = lax.axis_index("s")
    w = sid * 2 + cid
    zero = jnp.zeros((16,), _f32)

    def zrow(i, _):
        for kk in range(D // 16):
            rowbuf[i, pl.ds(kk * 16, 16)] = zero
        return _
    lax.fori_loop(0, CH, zrow, 0)
    rows_per = NPAD // 16
    r0 = sid * rows_per
    for b in range(rows_per // CH):
        pltpu.sync_copy(rowbuf, acc.at[pl.ds(r0 + b * CH, CH)])
    plsc.subcore_barrier()

    def chunk(j, _):
        sl_e = pl.ds(j * CH, CH)
        pltpu.sync_copy(gidxD.at[w].at[sl_e], gvc)
        pltpu.sync_copy(dstDf.at[w].at[sl_e], dvc)
        pltpu.sync_copy(coefD.at[w].at[sl_e], cvc)
        pltpu.async_copy(tbl.at[gvc], rowbuf, sem).wait()

        def grp(g, __):
            cvec = cvc[pl.ds(g * 16, 16)]
            for i in range(16):
                splat = _reg_splat(cvec, i)
                e = g * 16 + i
                for kk in range(D // 16):
                    sl = pl.ds(kk * 16, 16)
                    rowbuf[e, sl] = rowbuf[e, sl] * splat
            return __
        lax.fori_loop(0, CH // 16, grp, 0)
        pltpu.sync_copy(rowbuf, acc.at[dvc], add=True)
        return _
    lax.fori_loop(0, NCHD, chunk, 0)
    plsc.subcore_barrier()
    for b in range(rows_per // CH):
        sl = pl.ds(r0 + b * CH, CH)
        pltpu.sync_copy(acc.at[sl], outp.at[cid].at[sl])


@functools.lru_cache(maxsize=1)
def _sc_kernels():
    mesh = plsc.VectorSubcoreMesh(core_axis_name="c", subcore_axis_name="s")
    edge_prep = pl.kernel(
        _sc_edge_prep_body, mesh=mesh,
        out_type=[
            jax.ShapeDtypeStruct((EA, D), _f32),
            jax.ShapeDtypeStruct((NW, LA), _i32),
            jax.ShapeDtypeStruct((NW, LA), _i32),
        ],
        scratch_types=[
            pltpu.VMEM((LA,), _i32),
            pltpu.VMEM((LA,), _i32),
            pltpu.VMEM((LA,), _f32),
            pltpu.VMEM((LA,), _f32),
            pltpu.VMEM((LA,), _i32),
            pltpu.VMEM((LA,), _i32),
            pltpu.VMEM((CH, D), _f32),
            pltpu.SemaphoreType.DMA,
        ],
    )
    deg = pl.kernel(
        _sc_deg_body, mesh=mesh,
        out_type=[jax.ShapeDtypeStruct((2, NN2), _f32)],
        scratch_types=[
            pltpu.VMEM((LD,), _i32),
            pltpu.VMEM((LD,), _f32),
            pltpu.VMEM((CH,), _f32),
            pltpu.VMEM_SHARED((NN2,), _f32),
        ],
    )
    coef = pl.kernel(
        _sc_coef_body, mesh=mesh,
        out_type=[jax.ShapeDtypeStruct((NW, LD), _f32)],
        scratch_types=[
            pltpu.VMEM((LD,), _i32),
            pltpu.VMEM((LD,), _i32),
            pltpu.VMEM((LD,), _f32),
            pltpu.VMEM((LD,), _f32),
            pltpu.VMEM((LD,), _f32),
            pltpu.SemaphoreType.DMA,
        ],
    )
    scatter = pl.kernel(
        _sc_scatter_body, mesh=mesh,
        out_type=[jax.ShapeDtypeStruct((2, NPAD, D), _f32)],
        scratch_types=[
            pltpu.VMEM((CH,), _i32),
            pltpu.VMEM((CH,), _i32),
            pltpu.VMEM((CH,), _f32),
            pltpu.VMEM((CH, D), _f32),
            pltpu.VMEM_SHARED((NPAD, D), _f32),
            pltpu.SemaphoreType.DMA,
        ],
    )
    return edge_prep, deg, coef, scatter


def kernel(x, edge_index, edge_weight, batch, eeg_mask, eeg_W, eeg_b, fnirs_W, fnirs_b, em_W1, em_b1, em_W2, em_b2, intra_W0, intra_b0, cross_W0, cross_b0, gate0, bn_g0, bn_b0, intra_W1, intra_b1, cross_W1, cross_b1, gate1, bn_g1, bn_b1, co_W1, co_b1, co_W2, co_b2, g_W1, g_b1, g_W2, g_b2, cls_W1, cls_b1, cls_W2, cls_b2, pe_W1, pe_b1, pe_W2, pe_b2, pf_W1, pf_b1, pf_W2, pf_b2):
    src = edge_index[0]
    dst = edge_index[1]
    mf = eeg_mask.astype(_f32)

    xp = jnp.zeros((NPAD, D), _f32).at[:N].set(x)
    mfp = jnp.zeros((NPAD, 1), _f32).at[:N, 0].set(mf)
    mi2 = jnp.zeros((NN2,), _f32).at[:N].set(mf).at[NPAD:NPAD + N].set(mf)
    srcA = jnp.zeros((EA,), _i32).at[:E].set(src).reshape(NW, LA)
    dstnA = jnp.full((EA,), NPAD, _i32).at[:E].set(dst + NPAD).reshape(NW, LA)
    ewA = jnp.zeros((EA, 1), _f32).at[:E, 0].set(edge_weight)

    td = pl.pallas_call(
        _proj_body,
        grid=(NPAD // 512,),
        in_specs=[pl.BlockSpec((512, D), lambda i: (i, 0)),
                  pl.BlockSpec((512, 1), lambda i: (i, 0)),
                  _full((H, D)), _full((H,)), _full((H, D)), _full((H,))],
        out_specs=pl.BlockSpec((2, 512, D), lambda i: (0, i, 0)),
        out_shape=jax.ShapeDtypeStruct((2, NPAD, D), _f32),
    )(xp, mfp, eeg_W, eeg_b, fnirs_W, fnirs_b)
    h0 = td[0]
    td2 = td.reshape(2 * NPAD, D)

    _sc_edge_prep, _sc_deg, _sc_coef, _sc_scatter = _sc_kernels()
    sdiff, gidxA, ridxA = _sc_edge_prep(td2, srcA, dstnA, mi2)

    w1d = jnp.zeros((H, D), _f32).at[:64].set(em_W1[:, :D])
    w1l = jnp.zeros((1, H), _f32).at[0, :64].set(em_W1[:, D])
    b1p = jnp.zeros((1, H), _f32).at[0, :64].set(em_b1)
    a_em = jnp.zeros((H, H), _f32).at[:64, :].set(
        jnp.broadcast_to(em_W2[0][:, None], (64, H)))
    b2p = jnp.broadcast_to(em_b2[:, None], (1, H))
    ghat2d = pl.pallas_call(
        _edge_mlp_body,
        grid=(EA // 4096,),
        in_specs=[pl.BlockSpec((4096, D), lambda i: (i, 0)),
                  pl.BlockSpec((4096, 1), lambda i: (i, 0)),
                  _full((H, D)), _full((1, H)), _full((1, H)),
                  _full((H, H)), _full((1, H))],
        out_specs=pl.BlockSpec((4096, 1), lambda i: (i, 0)),
        out_shape=jax.ShapeDtypeStruct((EA, 1), _f32),
    )(sdiff, ewA, w1d, w1l, b1p, a_em, b2p)

    ghat_e = ghat2d[:, 0][:E]
    gidx_e = gidxA.reshape(-1)[:E]
    ridx_e = ridxA.reshape(-1)[:E]
    selfq = jnp.arange(NN2, dtype=_i32)
    selfd = jnp.concatenate([jnp.arange(NPAD, dtype=_i32)] * 2)
    npad_tail = ED - E - NN2
    gidxD = jnp.concatenate([gidx_e, selfq, jnp.zeros((npad_tail,), _i32)]).reshape(NW, LD)
    ridxD = jnp.concatenate([ridx_e, selfq, jnp.zeros((npad_tail,), _i32)]).reshape(NW, LD)
    dstDf = jnp.concatenate([dst, selfd, jnp.zeros((npad_tail,), _i32)]).reshape(NW, LD)
    ghatD = jnp.concatenate([ghat_e, jnp.ones((NN2,), _f32),
                             jnp.zeros((npad_tail,), _f32)]).reshape(NW, LD)

    degp = _sc_deg(ridxD, ghatD)[0]
    dinv2 = pl.pallas_call(
        _dinv_body,
        in_specs=[_full((2, NN2 // 128, 128))],
        out_specs=_full((NN2 // 128, 128)),
        out_shape=jax.ShapeDtypeStruct((NN2 // 128, 128), _f32),
    )(degp.reshape(2, NN2 // 128, 128)).reshape(NN2)
    coefD = _sc_coef(gidxD, ridxD, ghatD, dinv2)[0]

    bscale = 1.0 / jnp.sqrt(jnp.float32(1.0 + 1e-5))
    h = h0
    for (Wi, bi, Wc, bc, gt, bg, bb) in (
            (intra_W0, intra_b0, cross_W0, cross_b0, gate0, bn_g0, bn_b0),
            (intra_W1, intra_b1, cross_W1, cross_b1, gate1, bn_g1, bn_b1)):
        sig = jax.nn.sigmoid(gt[0])
        wcs = sig * Wc
        tbl = pl.pallas_call(
            _table_body,
            grid=(NPAD // 512,),
            in_specs=[pl.BlockSpec((512, D), lambda i: (i, 0)),
                      _full((H, D)), _full((H, D))],
            out_specs=pl.BlockSpec((2, 512, D), lambda i: (0, i, 0)),
            out_shape=jax.ShapeDtypeStruct((2, NPAD, D), _f32),
        )(h, Wi, wcs)
        outp = _sc_scatter(tbl.reshape(NN2, D), gidxD, dstDf, coefD)[0]
        scale_row = (bg * bscale)[None, :]
        bias_row = ((bi + sig * bc) * bg * bscale + bb)[None, :]
        h = pl.pallas_call(
            _combine_body,
            grid=(NPAD // 512,),
            in_specs=[pl.BlockSpec((2, 512, D), lambda i: (0, i, 0)),
                      _full((1, H)), _full((1, H))],
            out_specs=pl.BlockSpec((512, D), lambda i: (i, 0)),
            out_shape=jax.ShapeDtypeStruct((NPAD, D), _f32),
        )(outp, scale_row, bias_row)

    btp = jnp.full((NPAD,), B, _i32).at[:N].set(batch).reshape(NPAD // 512, 1, 512)
    mfrs = jnp.zeros((NPAD,), _f32).at[:N].set(mf).reshape(NPAD // 512, 1, 512)
    sa, se, ca, ce = pl.pallas_call(
        _pool_body,
        grid=(NPAD // 512,),
        in_specs=[pl.BlockSpec((512, D), lambda i: (i, 0)),
                  pl.BlockSpec((1, 1, 512), lambda i: (i, 0, 0)),
                  pl.BlockSpec((1, 1, 512), lambda i: (i, 0, 0))],
        out_specs=[_full((B, 128))] * 4,
        out_shape=[jax.ShapeDtypeStruct((B, 128), _f32)] * 4,
    )(h, btp, mfrs)

    co_A0 = jnp.broadcast_to(co_W2[0][:, None], (128, 128))
    co_A1 = jnp.broadcast_to(co_W2[1][:, None], (128, 128))
    co_b2r = jnp.broadcast_to(co_b2[:, None], (2, 128))
    g_A = jnp.broadcast_to(g_W2[0][:, None], (128, 128))
    g_b2r = jnp.broadcast_to(g_b2[:, None], (1, 128))
    cls_W1p = jnp.zeros((128, 128), _f32).at[:64, :].set(cls_W1)
    cls_b1p = jnp.zeros((128,), _f32).at[:64].set(cls_b1)
    cls_W2p = jnp.zeros((128, 128), _f32).at[:2, :64].set(cls_W2)
    cls_b2p = jnp.zeros((128,), _f32).at[:2].set(cls_b2)
    logits_p, ze, zf = pl.pallas_call(
        _head_body,
        out_shape=(jax.ShapeDtypeStruct((B, 128), _f32),
                   jax.ShapeDtypeStruct((B, 128), _f32),
                   jax.ShapeDtypeStruct((B, 128), _f32)),
    )(sa, se, ca, ce,
      co_W1, co_b1, co_A0, co_A1, co_b2r,
      g_W1, g_b1, g_A, g_b2r,
      cls_W1p, cls_b1p, cls_W2p, cls_b2p,
      pe_W1, pe_b1, pe_W2, pe_b2,
      pf_W1, pf_b1, pf_W2, pf_b2)
    return (logits_p[:, :2], ze, zf)
